# baseline (speedup 1.0000x reference)
import functools
import math

import jax
import jax.numpy as jnp
from jax.experimental import pallas as pl
from jax.experimental.pallas import tpu as pltpu

HID = 768
FF = 3072
E = 64
N_TOK = 2048
BLK = 128
NBLK = 80
NCAP = NBLK * BLK
_SQRT2 = math.sqrt(2.0)


def _cumsum_ax0(a):
    n = a.shape[0]
    k = 1
    while k < n:
        z = jnp.zeros((k, a.shape[1]), a.dtype)
        a = a + jnp.concatenate([z, a[: n - k]], axis=0)
        k *= 2
    return a


def _cumsum_ax1(a):
    n = a.shape[1]
    k = 1
    while k < n:
        z = jnp.zeros((a.shape[0], k), a.dtype)
        a = a + jnp.concatenate([z, a[:, : n - k]], axis=1)
        k *= 2
    return a


def _router_kernel(x_ref, gw_ref, noise_ref,
                   logits_ref, gating_ref, lp_ref, topi_ref, w_ref,
                   rid_ref, be_ref, nu_ref):
    x = x_ref[:]
    gw = gw_ref[:]
    logits = jax.lax.dot_general(
        x, gw, (((1,), (1,)), ((), ())), preferred_element_type=jnp.float32)
    logits_ref[:] = logits
    m = jnp.max(logits, axis=1, keepdims=True)
    ex = jnp.exp(logits - m)
    gating = ex / jnp.sum(ex, axis=1, keepdims=True)
    gating_ref[:] = gating
    noisy = logits + noise_ref[:]
    tau = jnp.max(noisy, axis=1, keepdims=True)
    z = (tau - logits) * float(E)
    lp_ref[:] = 0.5 * (1.0 - jax.lax.erf(z * (1.0 / _SQRT2)))
    iota_e = jax.lax.broadcasted_iota(jnp.int32, (N_TOK, E), 1)
    topi = jnp.min(jnp.where(logits == m, iota_e, E), axis=1, keepdims=True)
    topi_ref[:] = topi
    topw = jnp.max(gating, axis=1, keepdims=True)
    w_ref[:] = topw / (topw + 1e-9)
    mi = (topi == iota_e).astype(jnp.int32)
    counts = jnp.sum(mi, axis=0, keepdims=True)
    pc = ((counts + (BLK - 1)) // BLK) * BLK
    ends = _cumsum_ax1(pc)
    offs = ends - pc
    csum = _cumsum_ax0(mi)
    rank = jnp.sum(csum * mi, axis=1, keepdims=True) - 1
    rid_ref[:] = jnp.sum(mi * offs, axis=1, keepdims=True) + rank
    bstart = jax.lax.broadcasted_iota(jnp.int32, (NBLK, E), 0) * BLK
    ends_b = jnp.broadcast_to(ends, (NBLK, E))
    be = jnp.sum((ends_b <= bstart).astype(jnp.int32), axis=1, keepdims=True)
    be_ref[:] = jnp.minimum(be, E - 1)
    nu_ref[:] = ends[:, E - 1:E]


NFF = 2
FFC = FF // NFF


def _gmm_kernel(be_ref, nu_ref, x_ref, rid_ref, w_ref,
                w1_ref, b1_ref, w2_ref, b2_ref,
                out_ref, xb_scr, y_scr):
    b = pl.program_id(0)
    f = pl.program_id(1)

    @pl.when((b == 0) & (f == 0))
    def _():
        out_ref[:] = jnp.zeros_like(out_ref)

    @pl.when(b * BLK < nu_ref[0])
    def _():
        cmp = rid_ref[:] == (b * BLK
                             + jax.lax.broadcasted_iota(jnp.int32, (1, BLK), 1))

        @pl.when(f == 0)
        def _():
            m01 = cmp.astype(jnp.bfloat16)
            xb_scr[:] = jax.lax.dot_general(
                m01, x_ref[:], (((0,), (0,)), ((), ())),
                preferred_element_type=jnp.float32)

        h = jax.lax.dot_general(
            xb_scr[:], w1_ref[0], (((1,), (1,)), ((), ())),
            preferred_element_type=jnp.float32)
        h = h + b1_ref[0]
        h = 0.5 * h * (1.0 + jax.lax.erf(h * (1.0 / _SQRT2)))
        part = jax.lax.dot_general(
            h, w2_ref[0], (((1,), (1,)), ((), ())),
            preferred_element_type=jnp.float32)

        @pl.when(f == 0)
        def _():
            y_scr[:] = part + b2_ref[0]

        @pl.when(f == NFF - 1)
        def _():
            y16 = (y_scr[:] + part).astype(jnp.bfloat16)
            mw = jnp.where(cmp, w_ref[:], 0.0).astype(jnp.bfloat16)
            out_ref[:] += jax.lax.dot_general(
                mw, y16, (((1,), (0,)), ((), ())),
                preferred_element_type=jnp.float32)


def kernel(x, gate_w, fc1_w, fc1_b, fc2_w, fc2_b):
    B, S, D = x.shape
    xf = x.reshape(S, D)
    noise = jax.random.normal(jax.random.key(42), (S, E), dtype=jnp.float32) * (1.0 / E)

    f32 = jnp.float32
    i32 = jnp.int32
    logits, gating, lp, topi, w, rid, be, nu = pl.pallas_call(
        _router_kernel,
        out_shape=[
            jax.ShapeDtypeStruct((S, E), f32),
            jax.ShapeDtypeStruct((S, E), f32),
            jax.ShapeDtypeStruct((S, E), f32),
            jax.ShapeDtypeStruct((S, 1), i32),
            jax.ShapeDtypeStruct((S, 1), f32),
            jax.ShapeDtypeStruct((S, 1), i32),
            jax.ShapeDtypeStruct((NBLK, 1), i32),
            jax.ShapeDtypeStruct((1, 1), i32),
        ],
    )(xf, gate_w, noise)

    be1 = be.reshape(NBLK)
    nu1 = nu.reshape(1)

    out2d = pl.pallas_call(
        _gmm_kernel,
        grid_spec=pltpu.PrefetchScalarGridSpec(
            num_scalar_prefetch=2,
            grid=(NBLK, NFF),
            in_specs=[
                pl.BlockSpec((S, D), lambda b, f, be, nu: (0, 0)),
                pl.BlockSpec((S, 1), lambda b, f, be, nu: (0, 0)),
                pl.BlockSpec((S, 1), lambda b, f, be, nu: (0, 0)),
                pl.BlockSpec((1, FFC, D), lambda b, f, be, nu: (be[b], f, 0)),
                pl.BlockSpec((1, 1, FFC), lambda b, f, be, nu: (be[b], 0, f)),
                pl.BlockSpec((1, D, FFC), lambda b, f, be, nu: (be[b], 0, f)),
                pl.BlockSpec((1, 1, D), lambda b, f, be, nu: (be[b], 0, 0)),
            ],
            out_specs=pl.BlockSpec((S, D), lambda b, f, be, nu: (0, 0)),
            scratch_shapes=[
                pltpu.VMEM((BLK, D), f32),
                pltpu.VMEM((BLK, D), f32),
            ],
        ),
        out_shape=jax.ShapeDtypeStruct((S, D), f32),
    )(be1, nu1, xf.astype(jnp.bfloat16), rid, w,
      fc1_w, fc1_b.reshape(E, 1, FF), fc2_w, fc2_b.reshape(E, 1, D))

    output = out2d.reshape(B, S, D)
    return (output, gating, logits, lp, topi)


# speedup vs baseline: 1.2892x; 1.2892x over previous
import functools
import math

import jax
import jax.numpy as jnp
from jax.experimental import pallas as pl
from jax.experimental.pallas import tpu as pltpu

HID = 768
FF = 3072
E = 64
N_TOK = 2048
BLK = 128
NBLK = 80
NCAP = NBLK * BLK
_SQRT2 = math.sqrt(2.0)


def _cumsum_ax0(a):
    n = a.shape[0]
    k = 1
    while k < n:
        z = jnp.zeros((k, a.shape[1]), a.dtype)
        a = a + jnp.concatenate([z, a[: n - k]], axis=0)
        k *= 2
    return a


def _cumsum_ax1(a):
    n = a.shape[1]
    k = 1
    while k < n:
        z = jnp.zeros((a.shape[0], k), a.dtype)
        a = a + jnp.concatenate([z, a[:, : n - k]], axis=1)
        k *= 2
    return a


def _router_kernel(x_ref, gw_ref, noise_ref,
                   logits_ref, gating_ref, lp_ref, topi_ref, w_ref,
                   rid_ref, be_ref, nu_ref):
    x = x_ref[:]
    gw = gw_ref[:]
    logits = jax.lax.dot_general(
        x, gw, (((1,), (1,)), ((), ())), preferred_element_type=jnp.float32)
    logits_ref[:] = logits
    m = jnp.max(logits, axis=1, keepdims=True)
    ex = jnp.exp(logits - m)
    gating = ex / jnp.sum(ex, axis=1, keepdims=True)
    gating_ref[:] = gating
    noisy = logits + noise_ref[:]
    tau = jnp.max(noisy, axis=1, keepdims=True)
    z = (tau - logits) * float(E)
    lp_ref[:] = 0.5 * (1.0 - jax.lax.erf(z * (1.0 / _SQRT2)))
    iota_e = jax.lax.broadcasted_iota(jnp.int32, (N_TOK, E), 1)
    topi = jnp.min(jnp.where(logits == m, iota_e, E), axis=1, keepdims=True)
    topi_ref[:] = topi
    topw = jnp.max(gating, axis=1, keepdims=True)
    w_ref[:] = topw / (topw + 1e-9)
    mi = (topi == iota_e).astype(jnp.int32)
    counts = jnp.sum(mi, axis=0, keepdims=True)
    pc = ((counts + (BLK - 1)) // BLK) * BLK
    ends = _cumsum_ax1(pc)
    offs = ends - pc
    csum = _cumsum_ax0(mi)
    rank = jnp.sum(csum * mi, axis=1, keepdims=True) - 1
    rid_ref[:] = jnp.sum(mi * offs, axis=1, keepdims=True) + rank
    bstart = jax.lax.broadcasted_iota(jnp.int32, (NBLK, E), 0) * BLK
    ends_b = jnp.broadcast_to(ends, (NBLK, E))
    be = jnp.sum((ends_b <= bstart).astype(jnp.int32), axis=1, keepdims=True)
    be_ref[:] = jnp.minimum(be, E - 1)
    nu_ref[:] = ends[:, E - 1:E]


def _gmm_kernel(be_ref, nu_ref, x_ref, rid_ref,
                w1_ref, b1_ref, w2_ref, b2_ref, ys_ref):
    b = pl.program_id(0)

    @pl.when(b * BLK < nu_ref[0])
    def _():
        cmp = rid_ref[:] == (b * BLK
                             + jax.lax.broadcasted_iota(jnp.int32, (1, BLK), 1))
        m01 = cmp.astype(jnp.bfloat16)
        xb = jax.lax.dot_general(
            m01, x_ref[:], (((0,), (0,)), ((), ())),
            preferred_element_type=jnp.float32)
        h = jax.lax.dot_general(
            xb, w1_ref[0], (((1,), (1,)), ((), ())),
            preferred_element_type=jnp.float32)
        h = h + b1_ref[0]
        h = 0.5 * h * (1.0 + jax.lax.erf(h * (1.0 / _SQRT2)))
        y = jax.lax.dot_general(
            h, w2_ref[0], (((1,), (1,)), ((), ())),
            preferred_element_type=jnp.float32)
        ys_ref[:] = y + b2_ref[0]


def _unsort_kernel(rid_ref, ys_ref, w_ref, out_ref):
    def body(i, c):
        r = rid_ref[i]
        out_ref[pl.ds(i, 1), :] = ys_ref[pl.ds(r, 1), :]
        return c
    jax.lax.fori_loop(0, N_TOK, body, 0)
    out_ref[:] = out_ref[:] * w_ref[:]


def kernel(x, gate_w, fc1_w, fc1_b, fc2_w, fc2_b):
    B, S, D = x.shape
    xf = x.reshape(S, D)
    noise = jax.random.normal(jax.random.key(42), (S, E), dtype=jnp.float32) * (1.0 / E)

    f32 = jnp.float32
    i32 = jnp.int32
    logits, gating, lp, topi, w, rid, be, nu = pl.pallas_call(
        _router_kernel,
        out_shape=[
            jax.ShapeDtypeStruct((S, E), f32),
            jax.ShapeDtypeStruct((S, E), f32),
            jax.ShapeDtypeStruct((S, E), f32),
            jax.ShapeDtypeStruct((S, 1), i32),
            jax.ShapeDtypeStruct((S, 1), f32),
            jax.ShapeDtypeStruct((S, 1), i32),
            jax.ShapeDtypeStruct((NBLK, 1), i32),
            jax.ShapeDtypeStruct((1, 1), i32),
        ],
    )(xf, gate_w, noise)

    be1 = be.reshape(NBLK)
    nu1 = nu.reshape(1)
    rid1 = rid.reshape(S)

    ys = pl.pallas_call(
        _gmm_kernel,
        grid_spec=pltpu.PrefetchScalarGridSpec(
            num_scalar_prefetch=2,
            grid=(NBLK,),
            in_specs=[
                pl.BlockSpec((S, D), lambda b, be, nu: (0, 0)),
                pl.BlockSpec((S, 1), lambda b, be, nu: (0, 0)),
                pl.BlockSpec((1, FF, D), lambda b, be, nu: (be[b], 0, 0)),
                pl.BlockSpec((1, 1, FF), lambda b, be, nu: (be[b], 0, 0)),
                pl.BlockSpec((1, D, FF), lambda b, be, nu: (be[b], 0, 0)),
                pl.BlockSpec((1, 1, D), lambda b, be, nu: (be[b], 0, 0)),
            ],
            out_specs=pl.BlockSpec((BLK, D), lambda b, be, nu: (b, 0)),
        ),
        out_shape=jax.ShapeDtypeStruct((NCAP, D), f32),
    )(be1, nu1, xf.astype(jnp.bfloat16), rid,
      fc1_w, fc1_b.reshape(E, 1, FF), fc2_w, fc2_b.reshape(E, 1, D))

    out2d = pl.pallas_call(
        _unsort_kernel,
        grid_spec=pltpu.PrefetchScalarGridSpec(
            num_scalar_prefetch=1,
            grid=(1,),
            in_specs=[
                pl.BlockSpec((NCAP, D), lambda i, rid: (0, 0)),
                pl.BlockSpec((S, 1), lambda i, rid: (0, 0)),
            ],
            out_specs=pl.BlockSpec((S, D), lambda i, rid: (0, 0)),
        ),
        out_shape=jax.ShapeDtypeStruct((S, D), f32),
    )(rid1, ys, w)

    output = out2d.reshape(B, S, D)
    return (output, gating, logits, lp, topi)


# speedup vs baseline: 1.3530x; 1.0495x over previous
import functools
import math

import jax
import jax.numpy as jnp
from jax.experimental import pallas as pl
from jax.experimental.pallas import tpu as pltpu
from jax.experimental.pallas import tpu_sc as plsc

HID = 768
FF = 3072
E = 64
N_TOK = 2048
BLK = 128
NBLK = 80
NCAP = NBLK * BLK
_SQRT2 = math.sqrt(2.0)


def _cumsum_ax0(a):
    n = a.shape[0]
    k = 1
    while k < n:
        z = jnp.zeros((k, a.shape[1]), a.dtype)
        a = a + jnp.concatenate([z, a[: n - k]], axis=0)
        k *= 2
    return a


def _cumsum_ax1(a):
    n = a.shape[1]
    k = 1
    while k < n:
        z = jnp.zeros((a.shape[0], k), a.dtype)
        a = a + jnp.concatenate([z, a[:, : n - k]], axis=1)
        k *= 2
    return a


def _router_kernel(x_ref, gw_ref, noise_ref,
                   logits_ref, gating_ref, lp_ref, topi_ref, w_ref,
                   rid_ref, be_ref, nu_ref):
    x = x_ref[:]
    gw = gw_ref[:]
    logits = jax.lax.dot_general(
        x, gw, (((1,), (1,)), ((), ())), preferred_element_type=jnp.float32)
    logits_ref[:] = logits
    m = jnp.max(logits, axis=1, keepdims=True)
    ex = jnp.exp(logits - m)
    gating = ex / jnp.sum(ex, axis=1, keepdims=True)
    gating_ref[:] = gating
    noisy = logits + noise_ref[:]
    tau = jnp.max(noisy, axis=1, keepdims=True)
    z = (tau - logits) * float(E)
    lp_ref[:] = 0.5 * (1.0 - jax.lax.erf(z * (1.0 / _SQRT2)))
    iota_e = jax.lax.broadcasted_iota(jnp.int32, (N_TOK, E), 1)
    topi = jnp.min(jnp.where(logits == m, iota_e, E), axis=1, keepdims=True)
    topi_ref[:] = topi
    topw = jnp.max(gating, axis=1, keepdims=True)
    w_ref[:] = topw / (topw + 1e-9)
    mi = (topi == iota_e).astype(jnp.int32)
    counts = jnp.sum(mi, axis=0, keepdims=True)
    pc = ((counts + (BLK - 1)) // BLK) * BLK
    ends = _cumsum_ax1(pc)
    offs = ends - pc
    csum = _cumsum_ax0(mi)
    rank = jnp.sum(csum * mi, axis=1, keepdims=True) - 1
    rid_ref[:] = jnp.sum(mi * offs, axis=1, keepdims=True) + rank
    bstart = jax.lax.broadcasted_iota(jnp.int32, (NBLK, E), 0) * BLK
    ends_b = jnp.broadcast_to(ends, (NBLK, E))
    be = jnp.sum((ends_b <= bstart).astype(jnp.int32), axis=1, keepdims=True)
    be_ref[:] = jnp.minimum(be, E - 1)
    nu_ref[:] = ends[:, E - 1:E]


def _gmm_kernel(be_ref, nu_ref, x_ref, rid_ref, w_ref,
                w1_ref, b1_ref, w2_ref, b2_ref, ys_ref):
    b = pl.program_id(0)

    @pl.when(b * BLK < nu_ref[0])
    def _():
        cmp = rid_ref[:] == (b * BLK
                             + jax.lax.broadcasted_iota(jnp.int32, (1, BLK), 1))
        m01 = cmp.astype(jnp.bfloat16)
        xb = jax.lax.dot_general(
            m01, x_ref[:], (((0,), (0,)), ((), ())),
            preferred_element_type=jnp.float32)
        h = jax.lax.dot_general(
            xb, w1_ref[0], (((1,), (1,)), ((), ())),
            preferred_element_type=jnp.float32)
        h = h + b1_ref[0]
        h = 0.5 * h * (1.0 + jax.lax.erf(h * (1.0 / _SQRT2)))
        y = jax.lax.dot_general(
            h, w2_ref[0], (((1,), (1,)), ((), ())),
            preferred_element_type=jnp.float32)
        wb = jax.lax.dot_general(
            cmp.astype(jnp.float32), w_ref[:], (((0,), (0,)), ((), ())),
            preferred_element_type=jnp.float32)
        ys_ref[:] = (y + b2_ref[0]) * wb


_SC_NC = 2
_SC_NS = 16
_NW = _SC_NC * _SC_NS
_TPW = N_TOK // _NW


def _sc_unsort_body(ys_hbm, rid_hbm, out_hbm, idx_v, rows_v, sem):
    wid = jax.lax.axis_index("s") * _SC_NC + jax.lax.axis_index("c")
    base = wid * _TPW
    pltpu.sync_copy(rid_hbm.at[pl.ds(base, _TPW)], idx_v)
    pltpu.async_copy(ys_hbm.at[idx_v], rows_v, sem).wait()
    pltpu.sync_copy(rows_v, out_hbm.at[pl.ds(base, _TPW)])


def kernel(x, gate_w, fc1_w, fc1_b, fc2_w, fc2_b):
    B, S, D = x.shape
    xf = x.reshape(S, D)
    noise = jax.random.normal(jax.random.key(42), (S, E), dtype=jnp.float32) * (1.0 / E)

    f32 = jnp.float32
    i32 = jnp.int32
    logits, gating, lp, topi, w, rid, be, nu = pl.pallas_call(
        _router_kernel,
        out_shape=[
            jax.ShapeDtypeStruct((S, E), f32),
            jax.ShapeDtypeStruct((S, E), f32),
            jax.ShapeDtypeStruct((S, E), f32),
            jax.ShapeDtypeStruct((S, 1), i32),
            jax.ShapeDtypeStruct((S, 1), f32),
            jax.ShapeDtypeStruct((S, 1), i32),
            jax.ShapeDtypeStruct((NBLK, 1), i32),
            jax.ShapeDtypeStruct((1, 1), i32),
        ],
    )(xf, gate_w, noise)

    be1 = be.reshape(NBLK)
    nu1 = nu.reshape(1)
    rid1 = rid.reshape(S)

    ys = pl.pallas_call(
        _gmm_kernel,
        grid_spec=pltpu.PrefetchScalarGridSpec(
            num_scalar_prefetch=2,
            grid=(NBLK,),
            in_specs=[
                pl.BlockSpec((S, D), lambda b, be, nu: (0, 0)),
                pl.BlockSpec((S, 1), lambda b, be, nu: (0, 0)),
                pl.BlockSpec((S, 1), lambda b, be, nu: (0, 0)),
                pl.BlockSpec((1, FF, D), lambda b, be, nu: (be[b], 0, 0)),
                pl.BlockSpec((1, 1, FF), lambda b, be, nu: (be[b], 0, 0)),
                pl.BlockSpec((1, D, FF), lambda b, be, nu: (be[b], 0, 0)),
                pl.BlockSpec((1, 1, D), lambda b, be, nu: (be[b], 0, 0)),
            ],
            out_specs=pl.BlockSpec((BLK, D), lambda b, be, nu: (b, 0)),
        ),
        out_shape=jax.ShapeDtypeStruct((NCAP, D), f32),
    )(be1, nu1, xf.astype(jnp.bfloat16), rid, w,
      fc1_w, fc1_b.reshape(E, 1, FF), fc2_w, fc2_b.reshape(E, 1, D))

    out2d = pl.kernel(
        _sc_unsort_body,
        mesh=plsc.VectorSubcoreMesh(core_axis_name="c", subcore_axis_name="s"),
        out_type=jax.ShapeDtypeStruct((S, D), f32),
        scratch_types=[
            pltpu.VMEM((_TPW,), jnp.int32),
            pltpu.VMEM((_TPW, D), f32),
            pltpu.SemaphoreType.DMA,
        ],
    )(ys, rid1)

    output = out2d.reshape(B, S, D)
    return (output, gating, logits, lp, topi)
